# Initial kernel scaffold; baseline (speedup 1.0000x reference)
import jax
import jax.numpy as jnp
from jax.experimental import pallas as pl


def kernel(x1, x2, g_edge_index, pos_edge_index, neg_edge_index, W1_neigh, W1_self, b1, W2_neigh, W2_self, b2, mlp_W1, mlp_b1, bn_gamma, bn_beta, mlp_W2, mlp_b2):
    raise NotImplementedError("write your pallas kernel here")


# speedup vs baseline: 2.1562x; 2.1562x over previous
import functools

import jax
import jax.numpy as jnp
from jax import lax
from jax.experimental import pallas as pl
from jax.experimental.pallas import tpu as pltpu
from jax.experimental.pallas import tpu_sc as plsc

NC = 2
NS = 16
LN = 16
CHUNK = 128


def _cdiv(a, b):
    return (a + b - 1) // b


@functools.partial(jax.jit, static_argnums=(4, 5, 6))
def _seg_call(tab, srci, dsti, zrows, n_nodes, nch, want_deg):
    width = tab.shape[1]
    rps = zrows.shape[0]
    nacc = rps * NS

    def body(*refs):
        if want_deg:
            (tab_ref, src_ref, dst_ref, z_ref, zd_ref, out_ref, deg_ref,
             acc, degsh, idx_s, idx_d, rows, onesv, sem) = refs
        else:
            (tab_ref, src_ref, dst_ref, z_ref, zd_ref, out_ref,
             acc, idx_s, idx_d, rows, sem) = refs
        c = lax.axis_index("c")
        s = lax.axis_index("s")
        pltpu.sync_copy(z_ref, acc.at[pl.ds(s * rps, rps)])
        if want_deg:
            @pl.when(c == 0)
            def _():
                pltpu.sync_copy(zd_ref, degsh.at[pl.ds(s * rps, rps)])
                for i in range(CHUNK // LN):
                    onesv[pl.ds(i * LN, LN)] = jnp.full((LN,), 1.0, jnp.float32)
        plsc.subcore_barrier()
        off = c * n_nodes

        def step(j, carry):
            pltpu.sync_copy(src_ref.at[s, j], idx_s)
            pltpu.sync_copy(dst_ref.at[s, j], idx_d)
            for i in range(CHUNK // LN):
                sl = pl.ds(i * LN, LN)
                idx_s[sl] = idx_s[sl] + off
            pltpu.async_copy(tab_ref.at[idx_s], rows, sem).wait()
            if want_deg:
                @pl.when(c == 0)
                def _():
                    pltpu.sync_copy(onesv, degsh.at[idx_d], add=True)
            pltpu.sync_copy(rows, acc.at[idx_d], add=True)
            return carry

        lax.fori_loop(0, nch, step, 0)
        plsc.subcore_barrier()
        pltpu.sync_copy(acc.at[pl.ds(s * rps, rps)],
                        out_ref.at[c, pl.ds(s * rps, rps)])
        if want_deg:
            @pl.when(c == 0)
            def _():
                pltpu.sync_copy(degsh.at[pl.ds(s * rps, rps)],
                                deg_ref.at[pl.ds(s * rps, rps)])

    mesh = plsc.VectorSubcoreMesh(core_axis_name="c", subcore_axis_name="s")
    out_type = [jax.ShapeDtypeStruct((NC, nacc, width), jnp.float32)]
    scratch = [pltpu.VMEM_SHARED((nacc, width), jnp.float32)]
    if want_deg:
        out_type.append(jax.ShapeDtypeStruct((nacc,), jnp.float32))
        scratch.append(pltpu.VMEM_SHARED((nacc,), jnp.float32))
    scratch += [
        pltpu.VMEM((CHUNK,), jnp.int32),
        pltpu.VMEM((CHUNK,), jnp.int32),
        pltpu.VMEM((CHUNK, width), jnp.float32),
    ]
    if want_deg:
        scratch.append(pltpu.VMEM((CHUNK,), jnp.float32))
    scratch.append(pltpu.SemaphoreType.DMA)
    zdeg = jnp.zeros((rps,), jnp.float32)
    return pl.kernel(
        body,
        out_type=out_type if want_deg else out_type[0],
        mesh=mesh,
        scratch_types=scratch,
    )(tab, srci, dsti, zrows, zdeg)


@functools.partial(jax.jit, static_argnums=(3,))
def _edge_call(tab, ui, vi, nch):
    width = tab.shape[1]

    def body(tab_ref, u_ref, v_ref, out_ref,
             idx_u, idx_v, ru, rv, sem):
        c = lax.axis_index("c")
        s = lax.axis_index("s")
        w = s * NC + c

        def step(j, carry):
            pltpu.sync_copy(u_ref.at[w, j], idx_u)
            pltpu.sync_copy(v_ref.at[w, j], idx_v)
            pltpu.async_copy(tab_ref.at[idx_u], ru, sem).wait()
            pltpu.async_copy(tab_ref.at[idx_v], rv, sem).wait()

            def prow(r, cc):
                for i in range(width // LN):
                    sl = pl.ds(i * LN, LN)
                    ru[r, sl] = ru[r, sl] * rv[r, sl]
                return cc

            lax.fori_loop(0, CHUNK, prow, 0)
            pltpu.sync_copy(ru, out_ref.at[w, j])
            return carry

        lax.fori_loop(0, nch, step, 0)

    mesh = plsc.VectorSubcoreMesh(core_axis_name="c", subcore_axis_name="s")
    return pl.kernel(
        body,
        out_type=jax.ShapeDtypeStruct((NC * NS, nch, CHUNK, width), jnp.float32),
        mesh=mesh,
        scratch_types=[
            pltpu.VMEM((CHUNK,), jnp.int32),
            pltpu.VMEM((CHUNK,), jnp.int32),
            pltpu.VMEM((CHUNK, width), jnp.float32),
            pltpu.VMEM((CHUNK, width), jnp.float32),
            pltpu.SemaphoreType.DMA,
        ],
    )(tab, ui, vi)


@functools.partial(jax.jit, static_argnums=(6,))
def _sage_update(agg, x, degc, wn, ws, b, relu):
    n, d = x.shape[1], x.shape[2]
    bn = 1024

    def body(a_ref, x_ref, dg_ref, wn_ref, ws_ref, b_ref, o_ref):
        invd = 1.0 / jnp.maximum(dg_ref[...], 1.0)
        hn = a_ref[0] * invd
        h = (jnp.dot(hn, wn_ref[...], preferred_element_type=jnp.float32)
             + jnp.dot(x_ref[0], ws_ref[...], preferred_element_type=jnp.float32)
             + b_ref[...])
        if relu:
            h = jnp.maximum(h, 0.0)
        o_ref[0] = h

    return pl.pallas_call(
        body,
        grid=(2, n // bn),
        in_specs=[
            pl.BlockSpec((1, bn, d), lambda g, i: (g, i, 0)),
            pl.BlockSpec((1, bn, d), lambda g, i: (g, i, 0)),
            pl.BlockSpec((bn, 1), lambda g, i: (i, 0)),
            pl.BlockSpec((d, d), lambda g, i: (0, 0)),
            pl.BlockSpec((d, d), lambda g, i: (0, 0)),
            pl.BlockSpec((1, d), lambda g, i: (0, 0)),
        ],
        out_specs=pl.BlockSpec((1, bn, d), lambda g, i: (g, i, 0)),
        out_shape=jax.ShapeDtypeStruct((2, n, d), jnp.float32),
    )(agg, x, degc, wn, ws, b.reshape(1, d))


@functools.partial(jax.jit, static_argnums=(1, 2))
def _stats_call(prod, n_g, bn):
    d = prod.shape[1]

    def body(t_ref, s_ref, ts_ref):
        i = pl.program_id(0)

        @pl.when(i == 0)
        def _():
            s_ref[...] = jnp.zeros_like(s_ref)
            ts_ref[...] = jnp.zeros_like(ts_ref)

        t = t_ref[...]
        s_ref[...] += lax.dot_general(t, t, (((0,), (0,)), ((), ())),
                                      preferred_element_type=jnp.float32)
        ts_ref[...] += jnp.sum(t, axis=0, keepdims=True)

    return pl.pallas_call(
        body,
        grid=(n_g // bn,),
        in_specs=[pl.BlockSpec((bn, d), lambda i: (i, 0))],
        out_specs=[pl.BlockSpec((d, d), lambda i: (0, 0)),
                   pl.BlockSpec((1, d), lambda i: (0, 0))],
        out_shape=[jax.ShapeDtypeStruct((d, d), jnp.float32),
                   jax.ShapeDtypeStruct((1, d), jnp.float32)],
    )(prod)


@functools.partial(jax.jit, static_argnums=(5,))
def _bn_epilogue(s_mat, tsum, w1p, gam, bet, n_edges):
    d = s_mat.shape[0]

    def body(s_ref, ts_ref, w1_ref, g_ref, be_ref, sc_ref, sh_ref):
        w1 = w1_ref[...]
        t1 = jnp.dot(s_ref[...], w1, preferred_element_type=jnp.float32)
        diag_a = jnp.sum(t1 * w1, axis=0, keepdims=True)
        mu = jnp.dot(ts_ref[...] * (1.0 / n_edges), w1,
                     preferred_element_type=jnp.float32)
        var = diag_a * (1.0 / n_edges) - mu * mu
        inv = lax.rsqrt(var + 1e-5)
        sc = g_ref[...] * inv
        sc_ref[...] = sc
        sh_ref[...] = be_ref[...] - mu * sc

    return pl.pallas_call(
        body,
        out_shape=[jax.ShapeDtypeStruct((1, d), jnp.float32),
                   jax.ShapeDtypeStruct((1, d), jnp.float32)],
    )(s_mat, tsum, w1p, gam, bet)


@functools.partial(jax.jit, static_argnums=(6, 7))
def _pass2(prod, w1p, scale, shift, w2p, b2, n_g, bn):
    d = prod.shape[1]

    def body(t_ref, w1_ref, sc_ref, sh_ref, w2_ref, b2_ref, o_ref):
        y = jnp.dot(t_ref[...], w1_ref[...], preferred_element_type=jnp.float32)
        z = jnp.maximum(y * sc_ref[...] + sh_ref[...], 0.0)
        o_ref[...] = jnp.sum(z * w2_ref[...], axis=1, keepdims=True) + b2_ref[...]

    return pl.pallas_call(
        body,
        grid=(n_g // bn,),
        in_specs=[
            pl.BlockSpec((bn, d), lambda i: (i, 0)),
            pl.BlockSpec((d, d), lambda i: (0, 0)),
            pl.BlockSpec((1, d), lambda i: (0, 0)),
            pl.BlockSpec((1, d), lambda i: (0, 0)),
            pl.BlockSpec((1, d), lambda i: (0, 0)),
            pl.BlockSpec((1, 1), lambda i: (0, 0)),
        ],
        out_specs=pl.BlockSpec((bn, 1), lambda i: (i, 0)),
        out_shape=jax.ShapeDtypeStruct((n_g, 1), jnp.float32),
    )(prod, w1p, scale, shift, w2p, b2)


@functools.partial(jax.jit, static_argnums=(1, 2, 3))
def _rowsum(prod, n_rows, off_blocks, bn):
    d = prod.shape[1]

    def body(t_ref, o_ref):
        o_ref[...] = jnp.sum(t_ref[...], axis=1, keepdims=True)

    return pl.pallas_call(
        body,
        grid=(n_rows // bn,),
        in_specs=[pl.BlockSpec((bn, d), lambda i: (i + off_blocks, 0))],
        out_specs=pl.BlockSpec((bn, 1), lambda i: (i, 0)),
        out_shape=jax.ShapeDtypeStruct((n_rows, 1), jnp.float32),
    )(prod)


def kernel(x1, x2, g_edge_index, pos_edge_index, neg_edge_index,
           W1_neigh, W1_self, b1, W2_neigh, W2_self, b2,
           mlp_W1, mlp_b1, bn_gamma, bn_beta, mlp_W2, mlp_b2):
    n, d = x1.shape
    e = g_edge_index.shape[1]
    ep = pos_edge_index.shape[1]
    en = neg_edge_index.shape[1]
    mh = mlp_W1.shape[1]

    src, dst = g_edge_index[0], g_edge_index[1]

    nacc = _cdiv(n + 1, 1024) * 1024
    rps = nacc // NS

    zn = jnp.zeros((nacc - n, d), jnp.float32)
    xs = jnp.stack([jnp.concatenate([x1, zn], 0),
                    jnp.concatenate([x2, zn], 0)])

    nch_g = _cdiv(e, NS * CHUNK)
    epad = nch_g * NS * CHUNK
    srcp = jnp.concatenate([src, jnp.zeros((epad - e,), jnp.int32)]
                           ).reshape(NS, nch_g, CHUNK)
    dstp = jnp.concatenate([dst, jnp.full((epad - e,), n, jnp.int32)]
                           ).reshape(NS, nch_g, CHUNK)
    zrows = jnp.zeros((rps, d), jnp.float32)

    acc_a, deg = _seg_call(xs.reshape(2 * nacc, d), srcp, dstp, zrows,
                           nacc, nch_g, True)
    degc = deg.reshape(nacc, 1)
    h1 = _sage_update(acc_a, xs, degc, W1_neigh, W1_self, b1, True)

    acc_c = _seg_call(h1.reshape(2 * nacc, d), srcp, dstp, zrows,
                      nacc, nch_g, False)
    h = _sage_update(acc_c, h1, degc, W2_neigh, W2_self, b2, False)

    m = e + ep + en
    nch_e = _cdiv(m, NC * NS * CHUNK)
    mpad = nch_e * NC * NS * CHUNK
    u = jnp.concatenate([src, pos_edge_index[0], neg_edge_index[0],
                         jnp.zeros((mpad - m,), jnp.int32)]
                        ).reshape(NC * NS, nch_e, CHUNK)
    v = (jnp.concatenate([dst, pos_edge_index[1], neg_edge_index[1],
                          jnp.zeros((mpad - m,), jnp.int32)]) + nacc
         ).reshape(NC * NS, nch_e, CHUNK)
    prod = _edge_call(h.reshape(2 * nacc, d), u, v, nch_e).reshape(mpad, d)

    bn = 1000
    s_mat, tsum = _stats_call(prod, e, bn)
    w1p = jnp.zeros((d, d), jnp.float32).at[:, :mh].set(mlp_W1)
    gam = jnp.zeros((1, d), jnp.float32).at[0, :mh].set(bn_gamma)
    bet = jnp.zeros((1, d), jnp.float32).at[0, :mh].set(bn_beta)
    w2p = jnp.zeros((1, d), jnp.float32).at[0, :mh].set(mlp_W2[:, 0])
    scale, shift = _bn_epilogue(s_mat, tsum, w1p, gam, bet, float(e))
    rating = _pass2(prod, w1p, scale, shift, w2p,
                    mlp_b2.reshape(1, 1), e, bn)

    pn = _rowsum(prod, ep + en, e // bn, bn)
    return pn[:ep], pn[ep:ep + en], rating


# speedup vs baseline: 2.2615x; 1.0489x over previous
import functools

import jax
import jax.numpy as jnp
from jax import lax
from jax.experimental import pallas as pl
from jax.experimental.pallas import tpu as pltpu
from jax.experimental.pallas import tpu_sc as plsc

NC = 2
NS = 16
LN = 16
CHUNK = 128


def _cdiv(a, b):
    return (a + b - 1) // b


@functools.partial(jax.jit, static_argnums=(4, 5, 6))
def _seg_call(tab, srci, dsti, zrows, n_nodes, nch, want_deg):
    width = tab.shape[1]
    rps = zrows.shape[0]
    nacc = rps * NS

    def body(*refs):
        if want_deg:
            (tab_ref, src_ref, dst_ref, z_ref, zd_ref, out_ref, deg_ref,
             acc, degsh, idx_s0, idx_d0, rows0, idx_s1, idx_d1, rows1,
             onesv, sem0, sem1) = refs
        else:
            (tab_ref, src_ref, dst_ref, z_ref, zd_ref, out_ref,
             acc, idx_s0, idx_d0, rows0, idx_s1, idx_d1, rows1,
             sem0, sem1) = refs
        c = lax.axis_index("c")
        s = lax.axis_index("s")
        pltpu.sync_copy(z_ref, acc.at[pl.ds(s * rps, rps)])
        if want_deg:
            @pl.when(c == 0)
            def _():
                pltpu.sync_copy(zd_ref, degsh.at[pl.ds(s * rps, rps)])
                for i in range(CHUNK // LN):
                    onesv[pl.ds(i * LN, LN)] = jnp.full((LN,), 1.0, jnp.float32)
        plsc.subcore_barrier()
        off = c * n_nodes
        bufs = ((idx_s0, idx_d0, rows0, sem0), (idx_s1, idx_d1, rows1, sem1))

        def load_and_issue(j, bi_s, bi_d, br, bsem):
            pltpu.sync_copy(src_ref.at[s, j], bi_s)
            pltpu.sync_copy(dst_ref.at[s, j], bi_d)
            for i in range(CHUNK // LN):
                sl = pl.ds(i * LN, LN)
                bi_s[sl] = bi_s[sl] + off
            pltpu.async_copy(tab_ref.at[bi_s], br, bsem)

        load_and_issue(0, idx_s0, idx_d0, rows0, sem0)

        def pair(p, carry):
            j = p * 2
            for b in range(2):
                jj = j + b
                bi_s, bi_d, br, bsem = bufs[b]
                ni_s, ni_d, nr, nsem = bufs[1 - b]

                @pl.when(jj + 1 < nch)
                def _():
                    load_and_issue(jj + 1, ni_s, ni_d, nr, nsem)
                pltpu.make_async_copy(tab_ref.at[bi_s], br, bsem).wait()
                if want_deg:
                    @pl.when(c == 0)
                    def _():
                        pltpu.sync_copy(onesv, degsh.at[bi_d], add=True)
                pltpu.sync_copy(br, acc.at[bi_d], add=True)
            return carry

        lax.fori_loop(0, nch // 2, pair, 0)
        plsc.subcore_barrier()
        pltpu.sync_copy(acc.at[pl.ds(s * rps, rps)],
                        out_ref.at[c, pl.ds(s * rps, rps)])
        if want_deg:
            @pl.when(c == 0)
            def _():
                pltpu.sync_copy(degsh.at[pl.ds(s * rps, rps)],
                                deg_ref.at[pl.ds(s * rps, rps)])

    mesh = plsc.VectorSubcoreMesh(core_axis_name="c", subcore_axis_name="s")
    out_type = [jax.ShapeDtypeStruct((NC, nacc, width), jnp.float32)]
    scratch = [pltpu.VMEM_SHARED((nacc, width), jnp.float32)]
    if want_deg:
        out_type.append(jax.ShapeDtypeStruct((nacc,), jnp.float32))
        scratch.append(pltpu.VMEM_SHARED((nacc,), jnp.float32))
    scratch += [
        pltpu.VMEM((CHUNK,), jnp.int32),
        pltpu.VMEM((CHUNK,), jnp.int32),
        pltpu.VMEM((CHUNK, width), jnp.float32),
        pltpu.VMEM((CHUNK,), jnp.int32),
        pltpu.VMEM((CHUNK,), jnp.int32),
        pltpu.VMEM((CHUNK, width), jnp.float32),
    ]
    if want_deg:
        scratch.append(pltpu.VMEM((CHUNK,), jnp.float32))
    scratch += [pltpu.SemaphoreType.DMA, pltpu.SemaphoreType.DMA]
    zdeg = jnp.zeros((rps,), jnp.float32)
    return pl.kernel(
        body,
        out_type=out_type if want_deg else out_type[0],
        mesh=mesh,
        scratch_types=scratch,
    )(tab, srci, dsti, zrows, zdeg)


@functools.partial(jax.jit, static_argnums=(3,))
def _edge_call(tab, ui, vi, nch):
    width = tab.shape[1]

    def body(tab_ref, u_ref, v_ref, out_ref,
             iu0, iv0, ru0, rv0, iu1, iv1, ru1, rv1, sem0, sem1):
        c = lax.axis_index("c")
        s = lax.axis_index("s")
        w = s * NC + c
        bufs = ((iu0, iv0, ru0, rv0, sem0), (iu1, iv1, ru1, rv1, sem1))

        def load_and_issue(j, bu, bv, bru, brv, bsem):
            pltpu.sync_copy(u_ref.at[w, j], bu)
            pltpu.sync_copy(v_ref.at[w, j], bv)
            pltpu.async_copy(tab_ref.at[bu], bru, bsem)
            pltpu.async_copy(tab_ref.at[bv], brv, bsem)

        load_and_issue(0, iu0, iv0, ru0, rv0, sem0)

        def pair(p, carry):
            j = p * 2
            for b in range(2):
                jj = j + b
                bu, bv, bru, brv, bsem = bufs[b]
                nb = bufs[1 - b]

                @pl.when(jj + 1 < nch)
                def _():
                    load_and_issue(jj + 1, *nb)
                pltpu.make_async_copy(tab_ref.at[bu], bru, bsem).wait()
                pltpu.make_async_copy(tab_ref.at[bv], brv, bsem).wait()

                def prow(r, cc):
                    for i in range(width // LN):
                        sl = pl.ds(i * LN, LN)
                        bru[r, sl] = bru[r, sl] * brv[r, sl]
                    return cc

                lax.fori_loop(0, CHUNK, prow, 0)
                pltpu.sync_copy(bru, out_ref.at[w, jj])
            return carry

        lax.fori_loop(0, nch // 2, pair, 0)

    mesh = plsc.VectorSubcoreMesh(core_axis_name="c", subcore_axis_name="s")
    return pl.kernel(
        body,
        out_type=jax.ShapeDtypeStruct((NC * NS, nch, CHUNK, width), jnp.float32),
        mesh=mesh,
        scratch_types=[
            pltpu.VMEM((CHUNK,), jnp.int32),
            pltpu.VMEM((CHUNK,), jnp.int32),
            pltpu.VMEM((CHUNK, width), jnp.float32),
            pltpu.VMEM((CHUNK, width), jnp.float32),
            pltpu.VMEM((CHUNK,), jnp.int32),
            pltpu.VMEM((CHUNK,), jnp.int32),
            pltpu.VMEM((CHUNK, width), jnp.float32),
            pltpu.VMEM((CHUNK, width), jnp.float32),
            pltpu.SemaphoreType.DMA,
            pltpu.SemaphoreType.DMA,
        ],
    )(tab, ui, vi)


@functools.partial(jax.jit, static_argnums=(3,))
def _dot_call(tab, ui, vi, nch):
    width = tab.shape[1]

    def body(tab_ref, u_ref, v_ref, out_ref,
             iu0, iv0, ru0, rv0, iu1, iv1, ru1, rv1, ps, sem0, sem1):
        c = lax.axis_index("c")
        s = lax.axis_index("s")
        w = s * NC + c
        bufs = ((iu0, iv0, ru0, rv0, sem0), (iu1, iv1, ru1, rv1, sem1))

        def load_and_issue(j, bu, bv, bru, brv, bsem):
            pltpu.sync_copy(u_ref.at[w, j], bu)
            pltpu.sync_copy(v_ref.at[w, j], bv)
            pltpu.async_copy(tab_ref.at[bu], bru, bsem)
            pltpu.async_copy(tab_ref.at[bv], brv, bsem)

        load_and_issue(0, iu0, iv0, ru0, rv0, sem0)

        def pair(p, carry):
            j = p * 2
            for b in range(2):
                jj = j + b
                bu, bv, bru, brv, bsem = bufs[b]
                nb = bufs[1 - b]

                @pl.when(jj + 1 < nch)
                def _():
                    load_and_issue(jj + 1, *nb)
                pltpu.make_async_copy(tab_ref.at[bu], bru, bsem).wait()
                pltpu.make_async_copy(tab_ref.at[bv], brv, bsem).wait()

                def prow(r, cc):
                    acc16 = bru[r, pl.ds(0, LN)] * brv[r, pl.ds(0, LN)]
                    for i in range(1, width // LN):
                        sl = pl.ds(i * LN, LN)
                        acc16 = acc16 + bru[r, sl] * brv[r, sl]
                    ps[r, :] = acc16
                    return cc

                lax.fori_loop(0, CHUNK, prow, 0)
                pltpu.sync_copy(ps, out_ref.at[w, jj])
            return carry

        lax.fori_loop(0, nch // 2, pair, 0)

    mesh = plsc.VectorSubcoreMesh(core_axis_name="c", subcore_axis_name="s")
    return pl.kernel(
        body,
        out_type=jax.ShapeDtypeStruct((NC * NS, nch, CHUNK, LN), jnp.float32),
        mesh=mesh,
        scratch_types=[
            pltpu.VMEM((CHUNK,), jnp.int32),
            pltpu.VMEM((CHUNK,), jnp.int32),
            pltpu.VMEM((CHUNK, width), jnp.float32),
            pltpu.VMEM((CHUNK, width), jnp.float32),
            pltpu.VMEM((CHUNK,), jnp.int32),
            pltpu.VMEM((CHUNK,), jnp.int32),
            pltpu.VMEM((CHUNK, width), jnp.float32),
            pltpu.VMEM((CHUNK, width), jnp.float32),
            pltpu.VMEM((CHUNK, LN), jnp.float32),
            pltpu.SemaphoreType.DMA,
            pltpu.SemaphoreType.DMA,
        ],
    )(tab, ui, vi)


@functools.partial(jax.jit, static_argnums=(6,))
def _sage_update(agg, x, degc, wn, ws, b, relu):
    n, d = x.shape[1], x.shape[2]
    bn = 1024

    def body(a_ref, x_ref, dg_ref, wn_ref, ws_ref, b_ref, o_ref):
        invd = 1.0 / jnp.maximum(dg_ref[...], 1.0)
        hn = a_ref[0] * invd
        h = (jnp.dot(hn, wn_ref[...], preferred_element_type=jnp.float32)
             + jnp.dot(x_ref[0], ws_ref[...], preferred_element_type=jnp.float32)
             + b_ref[...])
        if relu:
            h = jnp.maximum(h, 0.0)
        o_ref[0] = h

    return pl.pallas_call(
        body,
        grid=(2, n // bn),
        in_specs=[
            pl.BlockSpec((1, bn, d), lambda g, i: (g, i, 0)),
            pl.BlockSpec((1, bn, d), lambda g, i: (g, i, 0)),
            pl.BlockSpec((bn, 1), lambda g, i: (i, 0)),
            pl.BlockSpec((d, d), lambda g, i: (0, 0)),
            pl.BlockSpec((d, d), lambda g, i: (0, 0)),
            pl.BlockSpec((1, d), lambda g, i: (0, 0)),
        ],
        out_specs=pl.BlockSpec((1, bn, d), lambda g, i: (g, i, 0)),
        out_shape=jax.ShapeDtypeStruct((2, n, d), jnp.float32),
    )(agg, x, degc, wn, ws, b.reshape(1, d))


@functools.partial(jax.jit, static_argnums=(1, 2))
def _stats_call(prod, n_g, bn):
    d = prod.shape[1]

    def body(t_ref, s_ref, ts_ref):
        i = pl.program_id(0)

        @pl.when(i == 0)
        def _():
            s_ref[...] = jnp.zeros_like(s_ref)
            ts_ref[...] = jnp.zeros_like(ts_ref)

        t = t_ref[...]
        s_ref[...] += lax.dot_general(t, t, (((0,), (0,)), ((), ())),
                                      preferred_element_type=jnp.float32)
        ts_ref[...] += jnp.sum(t, axis=0, keepdims=True)

    return pl.pallas_call(
        body,
        grid=(n_g // bn,),
        in_specs=[pl.BlockSpec((bn, d), lambda i: (i, 0))],
        out_specs=[pl.BlockSpec((d, d), lambda i: (0, 0)),
                   pl.BlockSpec((1, d), lambda i: (0, 0))],
        out_shape=[jax.ShapeDtypeStruct((d, d), jnp.float32),
                   jax.ShapeDtypeStruct((1, d), jnp.float32)],
    )(prod)


@functools.partial(jax.jit, static_argnums=(5,))
def _bn_epilogue(s_mat, tsum, w1p, gam, bet, n_edges):
    d = s_mat.shape[0]

    def body(s_ref, ts_ref, w1_ref, g_ref, be_ref, sc_ref, sh_ref):
        w1 = w1_ref[...]
        t1 = jnp.dot(s_ref[...], w1, preferred_element_type=jnp.float32)
        diag_a = jnp.sum(t1 * w1, axis=0, keepdims=True)
        mu = jnp.dot(ts_ref[...] * (1.0 / n_edges), w1,
                     preferred_element_type=jnp.float32)
        var = diag_a * (1.0 / n_edges) - mu * mu
        inv = lax.rsqrt(var + 1e-5)
        sc = g_ref[...] * inv
        sc_ref[...] = sc
        sh_ref[...] = be_ref[...] - mu * sc

    return pl.pallas_call(
        body,
        out_shape=[jax.ShapeDtypeStruct((1, d), jnp.float32),
                   jax.ShapeDtypeStruct((1, d), jnp.float32)],
    )(s_mat, tsum, w1p, gam, bet)


@functools.partial(jax.jit, static_argnums=(6, 7))
def _pass2(prod, w1p, scale, shift, w2p, b2, n_g, bn):
    d = prod.shape[1]

    def body(t_ref, w1_ref, sc_ref, sh_ref, w2_ref, b2_ref, o_ref):
        y = jnp.dot(t_ref[...], w1_ref[...], preferred_element_type=jnp.float32)
        z = jnp.maximum(y * sc_ref[...] + sh_ref[...], 0.0)
        o_ref[...] = jnp.sum(z * w2_ref[...], axis=1, keepdims=True) + b2_ref[...]

    return pl.pallas_call(
        body,
        grid=(n_g // bn,),
        in_specs=[
            pl.BlockSpec((bn, d), lambda i: (i, 0)),
            pl.BlockSpec((d, d), lambda i: (0, 0)),
            pl.BlockSpec((1, d), lambda i: (0, 0)),
            pl.BlockSpec((1, d), lambda i: (0, 0)),
            pl.BlockSpec((1, d), lambda i: (0, 0)),
            pl.BlockSpec((1, 1), lambda i: (0, 0)),
        ],
        out_specs=pl.BlockSpec((bn, 1), lambda i: (i, 0)),
        out_shape=jax.ShapeDtypeStruct((n_g, 1), jnp.float32),
    )(prod, w1p, scale, shift, w2p, b2)


@jax.jit
def _dotsum(psums):
    m = psums.shape[0]
    bn = 1024

    def body(t_ref, o_ref):
        o_ref[...] = jnp.sum(t_ref[...], axis=1, keepdims=True)

    return pl.pallas_call(
        body,
        grid=(m // bn,),
        in_specs=[pl.BlockSpec((bn, LN), lambda i: (i, 0))],
        out_specs=pl.BlockSpec((bn, 1), lambda i: (i, 0)),
        out_shape=jax.ShapeDtypeStruct((m, 1), jnp.float32),
    )(psums)


def kernel(x1, x2, g_edge_index, pos_edge_index, neg_edge_index,
           W1_neigh, W1_self, b1, W2_neigh, W2_self, b2,
           mlp_W1, mlp_b1, bn_gamma, bn_beta, mlp_W2, mlp_b2):
    n, d = x1.shape
    e = g_edge_index.shape[1]
    ep = pos_edge_index.shape[1]
    en = neg_edge_index.shape[1]
    mh = mlp_W1.shape[1]

    src, dst = g_edge_index[0], g_edge_index[1]

    nacc = _cdiv(n + 1, 1024) * 1024
    rps = nacc // NS

    zn = jnp.zeros((nacc - n, d), jnp.float32)
    xs = jnp.stack([jnp.concatenate([x1, zn], 0),
                    jnp.concatenate([x2, zn], 0)])

    nch_g = _cdiv(_cdiv(e, NS * CHUNK), 2) * 2
    epad = nch_g * NS * CHUNK
    srcp = jnp.concatenate([src, jnp.zeros((epad - e,), jnp.int32)]
                           ).reshape(NS, nch_g, CHUNK)
    dstp = jnp.concatenate([dst, jnp.full((epad - e,), n, jnp.int32)]
                           ).reshape(NS, nch_g, CHUNK)
    zrows = jnp.zeros((rps, d), jnp.float32)

    acc_a, deg = _seg_call(xs.reshape(2 * nacc, d), srcp, dstp, zrows,
                           nacc, nch_g, True)
    degc = deg.reshape(nacc, 1)
    h1 = _sage_update(acc_a, xs, degc, W1_neigh, W1_self, b1, True)

    acc_c = _seg_call(h1.reshape(2 * nacc, d), srcp, dstp, zrows,
                      nacc, nch_g, False)
    h = _sage_update(acc_c, h1, degc, W2_neigh, W2_self, b2, False)

    tab_h = h.reshape(2 * nacc, d)

    nch_e = _cdiv(_cdiv(e, NC * NS * CHUNK), 2) * 2
    mpad = nch_e * NC * NS * CHUNK
    ug = jnp.concatenate([src, jnp.zeros((mpad - e,), jnp.int32)]
                         ).reshape(NC * NS, nch_e, CHUNK)
    vg = (jnp.concatenate([dst, jnp.zeros((mpad - e,), jnp.int32)]) + nacc
          ).reshape(NC * NS, nch_e, CHUNK)
    prod = _edge_call(tab_h, ug, vg, nch_e).reshape(mpad, d)

    m2 = ep + en
    nch_d = _cdiv(_cdiv(m2, NC * NS * CHUNK), 2) * 2
    m2pad = nch_d * NC * NS * CHUNK
    up = jnp.concatenate([pos_edge_index[0], neg_edge_index[0],
                          jnp.zeros((m2pad - m2,), jnp.int32)]
                         ).reshape(NC * NS, nch_d, CHUNK)
    vp = (jnp.concatenate([pos_edge_index[1], neg_edge_index[1],
                           jnp.zeros((m2pad - m2,), jnp.int32)]) + nacc
          ).reshape(NC * NS, nch_d, CHUNK)
    psums = _dot_call(tab_h, up, vp, nch_d).reshape(m2pad, LN)
    pn = _dotsum(psums)

    bn = 1000
    s_mat, tsum = _stats_call(prod, e, bn)
    w1p = jnp.zeros((d, d), jnp.float32).at[:, :mh].set(mlp_W1)
    gam = jnp.zeros((1, d), jnp.float32).at[0, :mh].set(bn_gamma)
    bet = jnp.zeros((1, d), jnp.float32).at[0, :mh].set(bn_beta)
    w2p = jnp.zeros((1, d), jnp.float32).at[0, :mh].set(mlp_W2[:, 0])
    scale, shift = _bn_epilogue(s_mat, tsum, w1p, gam, bet, float(e))
    rating = _pass2(prod, w1p, scale, shift, w2p,
                    mlp_b2.reshape(1, 1), e, bn)

    return pn[:ep], pn[ep:ep + en], rating
